# Initial kernel scaffold; baseline (speedup 1.0000x reference)
import jax
import jax.numpy as jnp
from jax.experimental import pallas as pl


def kernel(index, length, data):
    raise NotImplementedError("write your pallas kernel here")


# speedup vs baseline: 2.8800x; 2.8800x over previous
import jax
import jax.numpy as jnp
from jax import lax
from jax.experimental import pallas as pl
from jax.experimental.pallas import tpu as pltpu
from jax.experimental.pallas import tpu_sc as plsc

_CYCLE = 168
_LEN = 336
_CH = 256
_B = 1024
_NC = 2
_NS = 16
_NW = _NC * _NS
_BPW = _B // _NW


def _sc_body(idx_hbm, data_hbm, out_hbm, idx_v, ddd_v, sem):
    wid = lax.axis_index("s") * _NC + lax.axis_index("c")
    base = wid * _BPW
    pltpu.sync_copy(idx_hbm.at[pl.ds(base, _BPW)], idx_v)
    pltpu.sync_copy(data_hbm, ddd_v.at[pl.ds(0, _CYCLE)])
    pltpu.sync_copy(data_hbm, ddd_v.at[pl.ds(_CYCLE, _CYCLE)])
    pltpu.sync_copy(data_hbm, ddd_v.at[pl.ds(2 * _CYCLE, _CYCLE)])
    copies = []
    for g in range(_BPW // 16):
        vec = idx_v[pl.ds(g * 16, 16)]
        for j in range(16):
            b = g * 16 + j
            i = vec[j]
            copies.append(
                pltpu.async_copy(ddd_v.at[pl.ds(i, _LEN)], out_hbm.at[base + b], sem)
            )
    for c in copies:
        c.wait()


def kernel(index, length, data):
    start = jnp.mod(index.astype(jnp.int32) + (length - _LEN), _CYCLE)
    start = start.astype(jnp.int32)
    mesh = plsc.VectorSubcoreMesh(core_axis_name="c", subcore_axis_name="s")
    k = pl.kernel(
        _sc_body,
        out_type=jax.ShapeDtypeStruct((_B, _LEN, _CH), jnp.float32),
        mesh=mesh,
        scratch_types=[
            pltpu.VMEM((_BPW,), jnp.int32),
            pltpu.VMEM((3 * _CYCLE, _CH), jnp.float32),
            pltpu.SemaphoreType.DMA,
        ],
        compiler_params=pltpu.CompilerParams(use_tc_tiling_on_sc=False),
    )
    return k(start, data)
